# baseline (speedup 1.0000x reference)
import functools

import jax
import jax.numpy as jnp
from jax import lax
from jax.experimental import pallas as pl
from jax.experimental.pallas import tpu as pltpu
from jax.experimental.pallas import tpu_sc as plsc

VOCAB = 100000
D = 64
N_NEGS = 20
B = 4096
C = 20

NC, NS, L = 2, 16, 16
NW = NC * NS
BPW = B // NW
NCOL = C + C * N_NEGS
JPAD = 512
NCH = JPAD // 128
RB = JPAD // L


def _sc_scores(cols, iword, emb_i, emb_o):
    mesh = plsc.VectorSubcoreMesh(core_axis_name="c", subcore_axis_name="s")

    @functools.partial(
        pl.kernel,
        mesh=mesh,
        compiler_params=pltpu.CompilerParams(
            needs_layout_passes=False, use_tc_tiling_on_sc=False,
            disable_bounds_checks=True),
        out_type=jax.ShapeDtypeStruct((B, JPAD), jnp.float32),
        scratch_types=[
            pltpu.VMEM((BPW,), jnp.int32),
            pltpu.VMEM((BPW, D), jnp.float32),
            pltpu.VMEM((JPAD,), jnp.int32),
            pltpu.VMEM((JPAD,), jnp.int32),
            pltpu.VMEM((JPAD, D), jnp.float32),
            pltpu.VMEM((JPAD, D), jnp.float32),
            pltpu.VMEM((D * L,), jnp.float32),
            pltpu.VMEM((JPAD,), jnp.float32),
            pltpu.SemaphoreType.DMA,
            pltpu.SemaphoreType.DMA,
        ],
    )
    def k(cols_hbm, iword_hbm, embi_hbm, embo_hbm, out_hbm,
          iwv, ivecs, idx0, idx1, rows0, rows1, bc, ov, sem0, sem1):
        wid = lax.axis_index("s") * NC + lax.axis_index("c")
        base = wid * BPW
        pltpu.sync_copy(iword_hbm.at[pl.ds(base, BPW)], iwv)
        pltpu.async_copy(embi_hbm.at[iwv], ivecs, sem0).wait()

        def fire(b, idxv, rows, sem):
            pltpu.sync_copy(cols_hbm.at[b], idxv)
            pltpu.async_copy(embo_hbm.at[idxv], rows, sem)

        def drain(idxv, rows, sem):
            pltpu.make_async_copy(embo_hbm.at[idxv], rows, sem).wait()

        def compute(rows, bi, b):
            zero_idx = jnp.zeros((L,), jnp.int32)
            bsplat = zero_idx + bi
            for d in range(D):
                bc[pl.ds(d * L, L)] = plsc.load_gather(
                    ivecs, [bsplat, jnp.full((L,), d, jnp.int32)])

            def rbody(rb, carry2):
                rowvec = rb * L + lax.iota(jnp.int32, L)
                acc = jnp.zeros((L,), jnp.float32)
                for d in range(D):
                    v = plsc.load_gather(
                        rows, [rowvec, jnp.full((L,), d, jnp.int32)])
                    acc = acc + v * bc[pl.ds(d * L, L)]
                ov[pl.ds(rb * L, L)] = acc
                return carry2

            lax.fori_loop(0, RB, rbody, 0)
            pltpu.sync_copy(ov, out_hbm.at[b])

        fire(base, idx0, rows0, sem0)

        def gbody(g, carry):
            b0 = base + 2 * g
            b1 = b0 + 1
            fire(b1, idx1, rows1, sem1)
            drain(idx0, rows0, sem0)
            compute(rows0, 2 * g, b0)

            @pl.when(g + 1 < BPW // 2)
            def _():
                fire(b0 + 2, idx0, rows0, sem0)

            drain(idx1, rows1, sem1)
            compute(rows1, 2 * g + 1, b1)
            return carry

        lax.fori_loop(0, BPW // 2, gbody, 0)

    return k(cols, iword, emb_i, emb_o)


def _tc_loss(scores):
    BLK = 256
    grid = B // BLK

    def body(s_ref, o_ref):
        i = pl.program_id(0)

        @pl.when(i == 0)
        def _():
            o_ref[0, 0] = 0.0

        t = s_ref[...]
        col = lax.broadcasted_iota(jnp.int32, (BLK, JPAD), 1)
        x = jnp.where(col < C, t, -t)
        ls = jnp.minimum(x, 0.0) - jnp.log1p(jnp.exp(-jnp.abs(x)))
        o_ref[0, 0] += jnp.sum(jnp.where(col < NCOL, ls, 0.0))

    out = pl.pallas_call(
        body,
        grid=(grid,),
        in_specs=[pl.BlockSpec((BLK, JPAD), lambda i: (i, 0))],
        out_specs=pl.BlockSpec(memory_space=pltpu.SMEM),
        out_shape=jax.ShapeDtypeStruct((1, 1), jnp.float32),
    )(scores)
    return -out[0, 0] / (B * C)


def kernel(iword, owords, emb_i, emb_o):
    nwords = jax.random.randint(
        jax.random.key(12345), (B, C * N_NEGS), 0, VOCAB - 1).astype(jnp.int32)
    cols = jnp.concatenate([owords, nwords], axis=1)
    cols = jnp.pad(cols, ((0, 0), (0, JPAD - NCOL)))
    scores = _sc_scores(cols, iword, emb_i, emb_o)
    return _tc_loss(scores)


# speedup vs baseline: 2.8865x; 2.8865x over previous
import functools

import jax
import jax.numpy as jnp
from jax import lax
from jax.experimental import pallas as pl
from jax.experimental.pallas import tpu as pltpu
from jax.experimental.pallas import tpu_sc as plsc

VOCAB = 100000
D = 64
N_NEGS = 20
B = 4096
C = 20

NC, NS, L = 2, 16, 16
NW = NC * NS
BPW = B // NW
NCOL = C + C * N_NEGS
JPAD = 512
JG = 448
RB = JG // L
NACC = 8


def _sc_scores(cols, iword, emb_i, emb_o):
    mesh = plsc.VectorSubcoreMesh(core_axis_name="c", subcore_axis_name="s")

    @functools.partial(
        pl.kernel,
        mesh=mesh,
        compiler_params=pltpu.CompilerParams(
            needs_layout_passes=False, use_tc_tiling_on_sc=False,
            disable_bounds_checks=True),
        out_type=jax.ShapeDtypeStruct((B, JPAD), jnp.float32),
        scratch_types=[
            pltpu.VMEM((BPW,), jnp.int32),
            pltpu.VMEM((BPW, D), jnp.float32),
            pltpu.VMEM((JG,), jnp.int32),
            pltpu.VMEM((JG,), jnp.int32),
            pltpu.VMEM((JG, D), jnp.float32),
            pltpu.VMEM((JG, D), jnp.float32),
            pltpu.VMEM((D * L,), jnp.float32),
            pltpu.VMEM((JPAD,), jnp.float32),
            pltpu.SemaphoreType.DMA,
            pltpu.SemaphoreType.DMA,
        ],
    )
    def k(cols_hbm, iword_hbm, embi_hbm, embo_hbm, out_hbm,
          iwv, ivecs, idx0, idx1, rows0, rows1, bc, ov, sem0, sem1):
        wid = lax.axis_index("s") * NC + lax.axis_index("c")
        base = wid * BPW
        pltpu.sync_copy(iword_hbm.at[pl.ds(base, BPW)], iwv)
        pltpu.async_copy(embi_hbm.at[iwv], ivecs, sem0).wait()

        def fire(b, idxv, rows, sem):
            pltpu.sync_copy(cols_hbm.at[b], idxv)
            pltpu.async_copy(embo_hbm.at[idxv], rows, sem)

        def drain(idxv, rows, sem):
            pltpu.make_async_copy(embo_hbm.at[idxv], rows, sem).wait()

        def compute(rows, bi, b):
            zero_idx = jnp.zeros((L,), jnp.int32)
            bsplat = zero_idx + bi
            for d in range(D):
                bc[pl.ds(d * L, L)] = plsc.load_gather(
                    ivecs, [bsplat, jnp.full((L,), d, jnp.int32)])

            def rbody(rb, carry2):
                rowvec = rb * L + lax.iota(jnp.int32, L)
                accs = [jnp.zeros((L,), jnp.float32) for _ in range(NACC)]
                for d in range(D):
                    v = plsc.load_gather(
                        rows, [rowvec, jnp.full((L,), d, jnp.int32)])
                    accs[d % NACC] = accs[d % NACC] + v * bc[pl.ds(d * L, L)]
                while len(accs) > 1:
                    accs = [a + b for a, b in zip(accs[::2], accs[1::2])]
                ov[pl.ds(rb * L, L)] = accs[0]
                return carry2

            lax.fori_loop(0, RB, rbody, 0)
            pltpu.sync_copy(ov, out_hbm.at[b])

        fire(base, idx0, rows0, sem0)

        def gbody(g, carry):
            b0 = base + 2 * g
            b1 = b0 + 1
            fire(b1, idx1, rows1, sem1)
            drain(idx0, rows0, sem0)
            compute(rows0, 2 * g, b0)

            @pl.when(g + 1 < BPW // 2)
            def _():
                fire(b0 + 2, idx0, rows0, sem0)

            drain(idx1, rows1, sem1)
            compute(rows1, 2 * g + 1, b1)
            return carry

        lax.fori_loop(0, BPW // 2, gbody, 0)

    return k(cols, iword, emb_i, emb_o)


def _tc_loss(scores):
    BLK = 256
    grid = B // BLK

    def body(s_ref, o_ref):
        i = pl.program_id(0)

        @pl.when(i == 0)
        def _():
            o_ref[0, 0] = 0.0

        t = s_ref[...]
        col = lax.broadcasted_iota(jnp.int32, (BLK, JPAD), 1)
        x = jnp.where(col < C, t, -t)
        ls = jnp.minimum(x, 0.0) - jnp.log1p(jnp.exp(-jnp.abs(x)))
        o_ref[0, 0] += jnp.sum(jnp.where(col < NCOL, ls, 0.0))

    out = pl.pallas_call(
        body,
        grid=(grid,),
        in_specs=[pl.BlockSpec((BLK, JPAD), lambda i: (i, 0))],
        out_specs=pl.BlockSpec(memory_space=pltpu.SMEM),
        out_shape=jax.ShapeDtypeStruct((1, 1), jnp.float32),
    )(scores)
    return -out[0, 0] / (B * C)


def kernel(iword, owords, emb_i, emb_o):
    nwords = jax.random.randint(
        jax.random.key(12345), (B, C * N_NEGS), 0, VOCAB - 1).astype(jnp.int32)
    cols = jnp.concatenate([owords, nwords], axis=1)
    cols = jnp.pad(cols, ((0, 0), (0, JG - NCOL)))
    scores = _sc_scores(cols, iword, emb_i, emb_o)
    return _tc_loss(scores)


# speedup vs baseline: 2.8890x; 1.0009x over previous
import functools

import jax
import jax.numpy as jnp
from jax import lax
from jax.experimental import pallas as pl
from jax.experimental.pallas import tpu as pltpu
from jax.experimental.pallas import tpu_sc as plsc

VOCAB = 100000
D = 64
N_NEGS = 20
B = 4096
C = 20

NC, NS, L = 2, 16, 16
NW = NC * NS
BPW = B // NW
NCOL = C + C * N_NEGS
JPAD = 512
JG = 448
RB = JG // L
NACC = 8


def _sc_scores(cols, iword, emb_i, emb_o):
    mesh = plsc.VectorSubcoreMesh(core_axis_name="c", subcore_axis_name="s")

    @functools.partial(
        pl.kernel,
        mesh=mesh,
        compiler_params=pltpu.CompilerParams(
            needs_layout_passes=False, use_tc_tiling_on_sc=False,
            disable_bounds_checks=True),
        out_type=jax.ShapeDtypeStruct((B, JPAD), jnp.float32),
        scratch_types=[
            pltpu.VMEM((BPW,), jnp.int32),
            pltpu.VMEM((BPW, D), jnp.float32),
            pltpu.VMEM((JG,), jnp.int32),
            pltpu.VMEM((JG,), jnp.int32),
            pltpu.VMEM((JG, D), jnp.float32),
            pltpu.VMEM((JG, D), jnp.float32),
            pltpu.VMEM((D * L,), jnp.float32),
            pltpu.VMEM((JPAD,), jnp.float32),
            pltpu.SemaphoreType.DMA,
            pltpu.SemaphoreType.DMA,
        ],
    )
    def k(cols_hbm, iword_hbm, embi_hbm, embo_hbm, out_hbm,
          iwv, ivecs, idx0, idx1, rows0, rows1, bc, ov, sem0, sem1):
        wid = lax.axis_index("s") * NC + lax.axis_index("c")
        base = wid * BPW
        pltpu.sync_copy(iword_hbm.at[pl.ds(base, BPW)], iwv)
        pltpu.async_copy(embi_hbm.at[iwv], ivecs, sem0).wait()

        def fire(b, idxv, rows, sem):
            pltpu.sync_copy(cols_hbm.at[b], idxv)
            pltpu.async_copy(embo_hbm.at[idxv], rows, sem)

        def drain(idxv, rows, sem):
            pltpu.make_async_copy(embo_hbm.at[idxv], rows, sem).wait()

        def compute(rows, bi, b):
            zero_idx = jnp.zeros((L,), jnp.int32)
            bsplat = zero_idx + bi
            for d in range(D):
                bc[pl.ds(d * L, L)] = plsc.load_gather(
                    ivecs, [bsplat, jnp.full((L,), d, jnp.int32)])

            @plsc.parallel_loop(0, RB, unroll=2)
            def rbody(rb):
                rowvec = rb * L + lax.iota(jnp.int32, L)
                accs = [jnp.zeros((L,), jnp.float32) for _ in range(NACC)]
                for d in range(D):
                    v = plsc.load_gather(
                        rows, [rowvec, jnp.full((L,), d, jnp.int32)])
                    accs[d % NACC] = accs[d % NACC] + v * bc[pl.ds(d * L, L)]
                while len(accs) > 1:
                    accs = [a + b for a, b in zip(accs[::2], accs[1::2])]
                ov[pl.ds(rb * L, L)] = accs[0]
            pltpu.sync_copy(ov, out_hbm.at[b])

        fire(base, idx0, rows0, sem0)

        def gbody(g, carry):
            b0 = base + 2 * g
            b1 = b0 + 1
            fire(b1, idx1, rows1, sem1)
            drain(idx0, rows0, sem0)
            compute(rows0, 2 * g, b0)

            @pl.when(g + 1 < BPW // 2)
            def _():
                fire(b0 + 2, idx0, rows0, sem0)

            drain(idx1, rows1, sem1)
            compute(rows1, 2 * g + 1, b1)
            return carry

        lax.fori_loop(0, BPW // 2, gbody, 0)

    return k(cols, iword, emb_i, emb_o)


def _tc_loss(scores):
    BLK = 256
    grid = B // BLK

    def body(s_ref, o_ref):
        i = pl.program_id(0)

        @pl.when(i == 0)
        def _():
            o_ref[0, 0] = 0.0

        t = s_ref[...]
        col = lax.broadcasted_iota(jnp.int32, (BLK, JPAD), 1)
        x = jnp.where(col < C, t, -t)
        ls = jnp.minimum(x, 0.0) - jnp.log1p(jnp.exp(-jnp.abs(x)))
        o_ref[0, 0] += jnp.sum(jnp.where(col < NCOL, ls, 0.0))

    out = pl.pallas_call(
        body,
        grid=(grid,),
        in_specs=[pl.BlockSpec((BLK, JPAD), lambda i: (i, 0))],
        out_specs=pl.BlockSpec(memory_space=pltpu.SMEM),
        out_shape=jax.ShapeDtypeStruct((1, 1), jnp.float32),
    )(scores)
    return -out[0, 0] / (B * C)


def kernel(iword, owords, emb_i, emb_o):
    nwords = jax.random.randint(
        jax.random.key(12345), (B, C * N_NEGS), 0, VOCAB - 1).astype(jnp.int32)
    cols = jnp.concatenate([owords, nwords], axis=1)
    cols = jnp.pad(cols, ((0, 0), (0, JG - NCOL)))
    scores = _sc_scores(cols, iword, emb_i, emb_o)
    return _tc_loss(scores)
